# baseline (speedup 1.0000x reference)
import functools

import jax
import jax.numpy as jnp
from jax.experimental import pallas as pl
from jax.experimental.pallas import tpu as pltpu

L, B, T, H, K, N, R = 4, 8, 128, 512, 4, 2048, 128


def _gather_body(heads_ref, tails_ref, relidx_ref, se_ref, rel_table_ref,
                 he_ref, te_ref, rel_ref):
    se = se_ref[0]
    heads = heads_ref[0]
    tails = tails_ref[0]
    relidx = relidx_ref[0]

    iota_n = jax.lax.broadcasted_iota(jnp.int32, (T, N), 1)
    acc_h = jnp.zeros((T, N), jnp.float32)
    acc_t = jnp.zeros((T, N), jnp.float32)
    for k in range(K):
        acc_h = acc_h + (heads[:, k:k + 1] == iota_n).astype(jnp.float32)
        acc_t = acc_t + (tails[:, k:k + 1] == iota_n).astype(jnp.float32)
    he_ref[0] = jnp.dot(acc_h, se, preferred_element_type=jnp.float32)
    te_ref[0] = jnp.dot(acc_t, se, preferred_element_type=jnp.float32)

    iota_r = jax.lax.broadcasted_iota(jnp.int32, (T, R), 1)
    onehot_r = (relidx.reshape(T, 1) == iota_r).astype(jnp.float32)
    rel_ref[0] = jnp.dot(onehot_r, rel_table_ref[...],
                         preferred_element_type=jnp.float32)


def _main_body(h_ref, he_ref, te_ref, rel_ref, mask_ref,
               w1_ref, w2_ref, bsq_ref, wq_ref, bq_ref, wk_ref, bk_ref,
               wd_ref, bd_ref, wg_ref, bg_ref, wa_ref, ba_ref, wu_ref, bu_ref,
               out_ref, u_scr, wd_scr, ug_scr, pasi_scr):
    h = h_ref[...]
    rel = rel_ref[...]
    m = mask_ref[...][:, 0, :]

    scale = 1.0 / (H ** 0.5)

    def mm(x, w):
        return jax.lax.dot_general(
            x, w, (((2,), (0,)), ((), ())),
            preferred_element_type=jnp.float32)

    q = mm(h, wq_ref[...]) + bq_ref[...]
    kk = mm(rel, wk_ref[...]) + bk_ref[...]
    scores = jax.lax.dot_general(
        q, kk, (((2,), (2,)), ((0,), (0,))),
        preferred_element_type=jnp.float32) * scale
    neg = jnp.where(m == 1, 0.0, -jnp.inf)
    scores = scores + neg[:, None, :]
    smax = jnp.max(scores, axis=-1, keepdims=True)
    e = jnp.exp(scores - smax)
    pasi_scr[...] = e / jnp.sum(e, axis=-1, keepdims=True)

    log2e = 1.4426950408889634
    wd = mm(h, wd_ref[...]) + bd_ref[...]
    wd_scr[...] = jnp.exp2((wd + bu_ref[...]) * (-log2e))
    ug_scr[...] = mm(h, wg_ref[...]) + bg_ref[...]

    u0 = mm(he_ref[...], w1_ref[...]) + mm(te_ref[...], w2_ref[...]) \
        + bsq_ref[...]
    mf = (m == 1).astype(jnp.float32)
    u_scr[...] = u0 * mf[:, :, None]

    h_last = h[:, T - 1:T, :]
    a_last = jax.nn.sigmoid(
        jax.lax.dot_general(h_last, wa_ref[...], (((2,), (0,)), ((), ())),
                            preferred_element_type=jnp.float32)
        + ba_ref[...])

    wu_vec = wu_ref[...] * (-log2e)

    def contract(tp, u):
        p = pasi_scr[:, pl.ds(tp, 1), :]
        out_ref[:, pl.ds(tp, 1), :] = jax.lax.dot_general(
            p, u, (((2,), (1,)), ((0,), (0,))),
            preferred_element_type=jnp.float32)

    def step(t, _):
        u = u_scr[...]
        contract(jnp.maximum(t - 1, 0), u)
        wu = jax.lax.dot_general(u, wu_vec, (((2,), (0,)), ((), ())),
                                 preferred_element_type=jnp.float32)
        ewu = jnp.exp2(wu)
        ent = wd_scr[:, pl.ds(t, 1), :]
        ugt = ug_scr[:, pl.ds(t, 1), :]
        bta = a_last / (1.0 + ent * ewu)
        un = u + bta * (ugt - u)
        u_scr[...] = un
        return 0

    jax.lax.fori_loop(0, T, step, 0, unroll=2)
    contract(T - 1, u_scr[...])


@functools.partial(jax.jit, static_argnames=("interpret",))
def kernel(batched_hidden_states, heads, tails, tri_mask, relations_idx,
           student_embeddings, rel_table, W_sq, b_sq, W_a, b_a, Wq, bq,
           Wk, bk, Wd, bd, Wu, bu, Wg, bg, interpret=False):
    h_last = batched_hidden_states[L - 1]
    relidx3 = relations_idx.reshape(B, 1, T).astype(jnp.int32)
    mask3 = tri_mask.reshape(B, 1, T).astype(jnp.int32)
    heads = heads.astype(jnp.int32)
    tails = tails.astype(jnp.int32)

    he, te, rel = pl.pallas_call(
        _gather_body,
        grid=(B,),
        in_specs=[
            pl.BlockSpec((1, T, K), lambda b: (b, 0, 0)),
            pl.BlockSpec((1, T, K), lambda b: (b, 0, 0)),
            pl.BlockSpec((1, 1, T), lambda b: (b, 0, 0)),
            pl.BlockSpec((1, N, H), lambda b: (b, 0, 0)),
            pl.BlockSpec((R, H), lambda b: (0, 0)),
        ],
        out_specs=[
            pl.BlockSpec((1, T, H), lambda b: (b, 0, 0)),
            pl.BlockSpec((1, T, H), lambda b: (b, 0, 0)),
            pl.BlockSpec((1, T, H), lambda b: (b, 0, 0)),
        ],
        out_shape=[jax.ShapeDtypeStruct((B, T, H), jnp.float32)] * 3,
        interpret=interpret,
    )(heads, tails, relidx3, student_embeddings, rel_table)

    w1 = W_sq[:, :H].T
    w2 = W_sq[:, H:].T

    out = pl.pallas_call(
        _main_body,
        in_specs=[
            pl.BlockSpec((B, T, H), lambda: (0, 0, 0)),
            pl.BlockSpec((B, T, H), lambda: (0, 0, 0)),
            pl.BlockSpec((B, T, H), lambda: (0, 0, 0)),
            pl.BlockSpec((B, T, H), lambda: (0, 0, 0)),
            pl.BlockSpec((B, 1, T), lambda: (0, 0, 0)),
            pl.BlockSpec((H, H), lambda: (0, 0)),
            pl.BlockSpec((H, H), lambda: (0, 0)),
            pl.BlockSpec((1, H), lambda: (0, 0)),
            pl.BlockSpec((H, H), lambda: (0, 0)),
            pl.BlockSpec((1, H), lambda: (0, 0)),
            pl.BlockSpec((H, H), lambda: (0, 0)),
            pl.BlockSpec((1, H), lambda: (0, 0)),
            pl.BlockSpec((H, H), lambda: (0, 0)),
            pl.BlockSpec((1, H), lambda: (0, 0)),
            pl.BlockSpec((H, H), lambda: (0, 0)),
            pl.BlockSpec((1, H), lambda: (0, 0)),
            pl.BlockSpec((H, 1), lambda: (0, 0)),
            pl.BlockSpec((1, 1), lambda: (0, 0)),
            pl.BlockSpec((H, 1), lambda: (0, 0)),
            pl.BlockSpec((1, 1), lambda: (0, 0)),
        ],
        out_specs=pl.BlockSpec((B, T, H), lambda: (0, 0, 0)),
        out_shape=jax.ShapeDtypeStruct((B, T, H), jnp.float32),
        scratch_shapes=[
            pltpu.VMEM((B, T, H), jnp.float32),
            pltpu.VMEM((B, T, H), jnp.float32),
            pltpu.VMEM((B, T, H), jnp.float32),
            pltpu.VMEM((B, T, T), jnp.float32),
        ],
        interpret=interpret,
    )(h_last, he, te, rel, mask3,
      w1, w2, b_sq.reshape(1, H), Wq.T, bq.reshape(1, H), Wk.T,
      bk.reshape(1, H), Wd.T, bd.reshape(1, H), Wg.T, bg.reshape(1, H),
      W_a.T, b_a.reshape(1, 1), Wu.T, bu.reshape(1, 1))
    return out


# speedup vs baseline: 1.1702x; 1.1702x over previous
import functools

import jax
import jax.numpy as jnp
from jax.experimental import pallas as pl
from jax.experimental.pallas import tpu as pltpu

L, B, T, H, K, N, R = 4, 8, 128, 512, 4, 2048, 128


def _gather_body(heads_ref, tails_ref, relidx_ref, se_ref, rel_table_ref,
                 he_ref, te_ref, rel_ref):
    se = se_ref[0]
    heads = heads_ref[0]
    tails = tails_ref[0]
    relidx = relidx_ref[0]

    iota_n = jax.lax.broadcasted_iota(jnp.int32, (T, N), 1)
    acc_h = jnp.zeros((T, N), jnp.float32)
    acc_t = jnp.zeros((T, N), jnp.float32)
    for k in range(K):
        acc_h = acc_h + (heads[:, k:k + 1] == iota_n).astype(jnp.float32)
        acc_t = acc_t + (tails[:, k:k + 1] == iota_n).astype(jnp.float32)
    he_ref[0] = jnp.dot(acc_h, se, preferred_element_type=jnp.float32)
    te_ref[0] = jnp.dot(acc_t, se, preferred_element_type=jnp.float32)

    iota_r = jax.lax.broadcasted_iota(jnp.int32, (T, R), 1)
    onehot_r = (relidx.reshape(T, 1) == iota_r).astype(jnp.float32)
    rel_ref[0] = jnp.dot(onehot_r, rel_table_ref[...],
                         preferred_element_type=jnp.float32)


def _main_body(h_ref, he_ref, te_ref, rel_ref, mask_ref,
               w1_ref, w2_ref, bsq_ref, wq_ref, bq_ref, wk_ref, bk_ref,
               wd_ref, bd_ref, wg_ref, bg_ref, wa_ref, ba_ref, wu_ref, bu_ref,
               out_ref, u_scr, wd_scr, ug_scr, pasi_scr):
    h = h_ref[...]
    rel = rel_ref[...]
    m = mask_ref[...][:, 0, :]

    scale = 1.0 / (H ** 0.5)

    def mm(x, w):
        return jax.lax.dot_general(
            x, w, (((2,), (0,)), ((), ())),
            preferred_element_type=jnp.float32)

    q = mm(h, wq_ref[...]) + bq_ref[...]
    kk = mm(rel, wk_ref[...]) + bk_ref[...]
    scores = jax.lax.dot_general(
        q, kk, (((2,), (2,)), ((0,), (0,))),
        preferred_element_type=jnp.float32) * scale
    neg = jnp.where(m == 1, 0.0, -jnp.inf)
    scores = scores + neg[:, None, :]
    smax = jnp.max(scores, axis=-1, keepdims=True)
    e = jnp.exp(scores - smax)
    pasi_scr[...] = e / jnp.sum(e, axis=-1, keepdims=True)

    log2e = 1.4426950408889634
    wd = mm(h, wd_ref[...]) + bd_ref[...]
    wd_scr[...] = jnp.exp2((wd + bu_ref[...]) * (-log2e))
    ug_scr[...] = mm(h, wg_ref[...]) + bg_ref[...]

    u0 = mm(he_ref[...], w1_ref[...]) + mm(te_ref[...], w2_ref[...]) \
        + bsq_ref[...]
    mf = (m == 1).astype(jnp.float32)
    u_scr[...] = u0 * mf[:, :, None]

    h_last = h[:, T - 1:T, :]
    a_last = jax.nn.sigmoid(
        jax.lax.dot_general(h_last, wa_ref[...], (((2,), (0,)), ((), ())),
                            preferred_element_type=jnp.float32)
        + ba_ref[...])

    wu_vec = wu_ref[...] * (-log2e)

    def contract(tp, u):
        p = pasi_scr[:, pl.ds(tp, 1), :]
        out_ref[:, pl.ds(tp, 1), :] = jax.lax.dot_general(
            p, u, (((2,), (1,)), ((0,), (0,))),
            preferred_element_type=jnp.float32)

    def matvec(x):
        return jax.lax.dot_general(x, wu_vec, (((2,), (0,)), ((), ())),
                                   preferred_element_type=jnp.float32)

    def step(t, wu):
        u = u_scr[...]
        contract(jnp.maximum(t - 1, 0), u)
        ewu = jnp.exp2(wu)
        ent = wd_scr[:, pl.ds(t, 1), :]
        ugt = ug_scr[:, pl.ds(t, 1), :]
        bta = a_last / (1.0 + ent * ewu)
        un = u + bta * (ugt - u)
        u_scr[...] = un
        return matvec(un)

    jax.lax.fori_loop(0, T, step, matvec(u_scr[...]))
    contract(T - 1, u_scr[...])


@functools.partial(jax.jit, static_argnames=("interpret",))
def kernel(batched_hidden_states, heads, tails, tri_mask, relations_idx,
           student_embeddings, rel_table, W_sq, b_sq, W_a, b_a, Wq, bq,
           Wk, bk, Wd, bd, Wu, bu, Wg, bg, interpret=False):
    h_last = batched_hidden_states[L - 1]
    relidx3 = relations_idx.reshape(B, 1, T).astype(jnp.int32)
    mask3 = tri_mask.reshape(B, 1, T).astype(jnp.int32)
    heads = heads.astype(jnp.int32)
    tails = tails.astype(jnp.int32)

    he, te, rel = pl.pallas_call(
        _gather_body,
        grid=(B,),
        in_specs=[
            pl.BlockSpec((1, T, K), lambda b: (b, 0, 0)),
            pl.BlockSpec((1, T, K), lambda b: (b, 0, 0)),
            pl.BlockSpec((1, 1, T), lambda b: (b, 0, 0)),
            pl.BlockSpec((1, N, H), lambda b: (b, 0, 0)),
            pl.BlockSpec((R, H), lambda b: (0, 0)),
        ],
        out_specs=[
            pl.BlockSpec((1, T, H), lambda b: (b, 0, 0)),
            pl.BlockSpec((1, T, H), lambda b: (b, 0, 0)),
            pl.BlockSpec((1, T, H), lambda b: (b, 0, 0)),
        ],
        out_shape=[jax.ShapeDtypeStruct((B, T, H), jnp.float32)] * 3,
        interpret=interpret,
    )(heads, tails, relidx3, student_embeddings, rel_table)

    w1 = W_sq[:, :H].T
    w2 = W_sq[:, H:].T

    out = pl.pallas_call(
        _main_body,
        in_specs=[
            pl.BlockSpec((B, T, H), lambda: (0, 0, 0)),
            pl.BlockSpec((B, T, H), lambda: (0, 0, 0)),
            pl.BlockSpec((B, T, H), lambda: (0, 0, 0)),
            pl.BlockSpec((B, T, H), lambda: (0, 0, 0)),
            pl.BlockSpec((B, 1, T), lambda: (0, 0, 0)),
            pl.BlockSpec((H, H), lambda: (0, 0)),
            pl.BlockSpec((H, H), lambda: (0, 0)),
            pl.BlockSpec((1, H), lambda: (0, 0)),
            pl.BlockSpec((H, H), lambda: (0, 0)),
            pl.BlockSpec((1, H), lambda: (0, 0)),
            pl.BlockSpec((H, H), lambda: (0, 0)),
            pl.BlockSpec((1, H), lambda: (0, 0)),
            pl.BlockSpec((H, H), lambda: (0, 0)),
            pl.BlockSpec((1, H), lambda: (0, 0)),
            pl.BlockSpec((H, H), lambda: (0, 0)),
            pl.BlockSpec((1, H), lambda: (0, 0)),
            pl.BlockSpec((H, 1), lambda: (0, 0)),
            pl.BlockSpec((1, 1), lambda: (0, 0)),
            pl.BlockSpec((H, 1), lambda: (0, 0)),
            pl.BlockSpec((1, 1), lambda: (0, 0)),
        ],
        out_specs=pl.BlockSpec((B, T, H), lambda: (0, 0, 0)),
        out_shape=jax.ShapeDtypeStruct((B, T, H), jnp.float32),
        scratch_shapes=[
            pltpu.VMEM((B, T, H), jnp.float32),
            pltpu.VMEM((B, T, H), jnp.float32),
            pltpu.VMEM((B, T, H), jnp.float32),
            pltpu.VMEM((B, T, T), jnp.float32),
        ],
        interpret=interpret,
    )(h_last, he, te, rel, mask3,
      w1, w2, b_sq.reshape(1, H), Wq.T, bq.reshape(1, H), Wk.T,
      bk.reshape(1, H), Wd.T, bd.reshape(1, H), Wg.T, bg.reshape(1, H),
      W_a.T, b_a.reshape(1, 1), Wu.T, bu.reshape(1, 1))
    return out


# speedup vs baseline: 1.2152x; 1.0384x over previous
import functools

import jax
import jax.numpy as jnp
from jax.experimental import pallas as pl
from jax.experimental.pallas import tpu as pltpu

L, B, T, H, K, N, R = 4, 8, 128, 512, 4, 2048, 128


def _gather_body(heads_ref, tails_ref, relidx_ref, se_ref, rel_table_ref,
                 he_ref, te_ref, rel_ref):
    se = se_ref[0]
    heads = heads_ref[0]
    tails = tails_ref[0]
    relidx = relidx_ref[0]

    iota_n = jax.lax.broadcasted_iota(jnp.int32, (T, N), 1)
    acc_h = jnp.zeros((T, N), jnp.float32)
    acc_t = jnp.zeros((T, N), jnp.float32)
    for k in range(K):
        acc_h = acc_h + (heads[:, k:k + 1] == iota_n).astype(jnp.float32)
        acc_t = acc_t + (tails[:, k:k + 1] == iota_n).astype(jnp.float32)
    he_ref[0] = jnp.dot(acc_h, se, preferred_element_type=jnp.float32)
    te_ref[0] = jnp.dot(acc_t, se, preferred_element_type=jnp.float32)

    iota_r = jax.lax.broadcasted_iota(jnp.int32, (T, R), 1)
    onehot_r = (relidx.reshape(T, 1) == iota_r).astype(jnp.float32)
    rel_ref[0] = jnp.dot(onehot_r, rel_table_ref[...],
                         preferred_element_type=jnp.float32)


def _main_body(h_ref, he_ref, te_ref, rel_ref, mask_ref,
               w1_ref, w2_ref, bsq_ref, wq_ref, bq_ref, wk_ref, bk_ref,
               wd_ref, bd_ref, wg_ref, bg_ref, wa_ref, ba_ref, wu_ref, bu_ref,
               out_ref, u_scr, wd_scr, ug_scr, pasi_scr):
    h = h_ref[...]
    rel = rel_ref[...]
    m = mask_ref[...][:, 0, :]

    scale = 1.0 / (H ** 0.5)

    def mm(x, w):
        return jax.lax.dot_general(
            x, w, (((2,), (0,)), ((), ())),
            preferred_element_type=jnp.float32)

    q = mm(h, wq_ref[...]) + bq_ref[...]
    kk = mm(rel, wk_ref[...]) + bk_ref[...]
    scores = jax.lax.dot_general(
        q, kk, (((2,), (2,)), ((0,), (0,))),
        preferred_element_type=jnp.float32) * scale
    neg = jnp.where(m == 1, 0.0, -jnp.inf)
    scores = scores + neg[:, None, :]
    smax = jnp.max(scores, axis=-1, keepdims=True)
    e = jnp.exp(scores - smax)
    pasi_scr[...] = e / jnp.sum(e, axis=-1, keepdims=True)

    log2e = 1.4426950408889634
    wd = mm(h, wd_ref[...]) + bd_ref[...]
    ug_scr[...] = mm(h, wg_ref[...]) + bg_ref[...]

    u0 = mm(he_ref[...], w1_ref[...]) + mm(te_ref[...], w2_ref[...]) \
        + bsq_ref[...]
    mf = (m == 1).astype(jnp.float32)
    u_scr[...] = u0 * mf[:, :, None]

    h_last = h[:, T - 1:T, :]
    a_last = jax.nn.sigmoid(
        jax.lax.dot_general(h_last, wa_ref[...], (((2,), (0,)), ((), ())),
                            preferred_element_type=jnp.float32)
        + ba_ref[...])
    inva = 1.0 / a_last
    wd_scr[...] = inva * jnp.exp2((wd + bu_ref[...]) * (-log2e))

    wu_vec = wu_ref[...] * (-log2e)

    def contract(tp, u):
        p = pasi_scr[:, pl.ds(tp, 1), :]
        out_ref[:, pl.ds(tp, 1), :] = jax.lax.dot_general(
            p, u, (((2,), (1,)), ((0,), (0,))),
            preferred_element_type=jnp.float32)

    def matvec(x):
        return jax.lax.dot_general(x, wu_vec, (((2,), (0,)), ((), ())),
                                   preferred_element_type=jnp.float32)

    def step(t, ewu):
        u = u_scr[...]
        contract(jnp.maximum(t - 1, 0), u)
        ent = wd_scr[:, pl.ds(t, 1), :]
        ugt = ug_scr[:, pl.ds(t, 1), :]
        bta = 1.0 / (inva + ent * ewu)
        un = u + bta * (ugt - u)
        u_scr[...] = un
        return jnp.exp2(matvec(un))

    jax.lax.fori_loop(0, T, step, jnp.exp2(matvec(u_scr[...])))
    contract(T - 1, u_scr[...])


@functools.partial(jax.jit, static_argnames=("interpret",))
def kernel(batched_hidden_states, heads, tails, tri_mask, relations_idx,
           student_embeddings, rel_table, W_sq, b_sq, W_a, b_a, Wq, bq,
           Wk, bk, Wd, bd, Wu, bu, Wg, bg, interpret=False):
    h_last = batched_hidden_states[L - 1]
    relidx3 = relations_idx.reshape(B, 1, T).astype(jnp.int32)
    mask3 = tri_mask.reshape(B, 1, T).astype(jnp.int32)
    heads = heads.astype(jnp.int32)
    tails = tails.astype(jnp.int32)

    he, te, rel = pl.pallas_call(
        _gather_body,
        grid=(B,),
        in_specs=[
            pl.BlockSpec((1, T, K), lambda b: (b, 0, 0)),
            pl.BlockSpec((1, T, K), lambda b: (b, 0, 0)),
            pl.BlockSpec((1, 1, T), lambda b: (b, 0, 0)),
            pl.BlockSpec((1, N, H), lambda b: (b, 0, 0)),
            pl.BlockSpec((R, H), lambda b: (0, 0)),
        ],
        out_specs=[
            pl.BlockSpec((1, T, H), lambda b: (b, 0, 0)),
            pl.BlockSpec((1, T, H), lambda b: (b, 0, 0)),
            pl.BlockSpec((1, T, H), lambda b: (b, 0, 0)),
        ],
        out_shape=[jax.ShapeDtypeStruct((B, T, H), jnp.float32)] * 3,
        interpret=interpret,
    )(heads, tails, relidx3, student_embeddings, rel_table)

    w1 = W_sq[:, :H].T
    w2 = W_sq[:, H:].T

    out = pl.pallas_call(
        _main_body,
        in_specs=[
            pl.BlockSpec((B, T, H), lambda: (0, 0, 0)),
            pl.BlockSpec((B, T, H), lambda: (0, 0, 0)),
            pl.BlockSpec((B, T, H), lambda: (0, 0, 0)),
            pl.BlockSpec((B, T, H), lambda: (0, 0, 0)),
            pl.BlockSpec((B, 1, T), lambda: (0, 0, 0)),
            pl.BlockSpec((H, H), lambda: (0, 0)),
            pl.BlockSpec((H, H), lambda: (0, 0)),
            pl.BlockSpec((1, H), lambda: (0, 0)),
            pl.BlockSpec((H, H), lambda: (0, 0)),
            pl.BlockSpec((1, H), lambda: (0, 0)),
            pl.BlockSpec((H, H), lambda: (0, 0)),
            pl.BlockSpec((1, H), lambda: (0, 0)),
            pl.BlockSpec((H, H), lambda: (0, 0)),
            pl.BlockSpec((1, H), lambda: (0, 0)),
            pl.BlockSpec((H, H), lambda: (0, 0)),
            pl.BlockSpec((1, H), lambda: (0, 0)),
            pl.BlockSpec((H, 1), lambda: (0, 0)),
            pl.BlockSpec((1, 1), lambda: (0, 0)),
            pl.BlockSpec((H, 1), lambda: (0, 0)),
            pl.BlockSpec((1, 1), lambda: (0, 0)),
        ],
        out_specs=pl.BlockSpec((B, T, H), lambda: (0, 0, 0)),
        out_shape=jax.ShapeDtypeStruct((B, T, H), jnp.float32),
        scratch_shapes=[
            pltpu.VMEM((B, T, H), jnp.float32),
            pltpu.VMEM((B, T, H), jnp.float32),
            pltpu.VMEM((B, T, H), jnp.float32),
            pltpu.VMEM((B, T, T), jnp.float32),
        ],
        interpret=interpret,
    )(h_last, he, te, rel, mask3,
      w1, w2, b_sq.reshape(1, H), Wq.T, bq.reshape(1, H), Wk.T,
      bk.reshape(1, H), Wd.T, bd.reshape(1, H), Wg.T, bg.reshape(1, H),
      W_a.T, b_a.reshape(1, 1), Wu.T, bu.reshape(1, 1))
    return out
